# baseline (speedup 1.0000x reference)
import functools

import jax
import jax.numpy as jnp
from jax import lax
from jax.experimental import pallas as pl
from jax.experimental.pallas import tpu as pltpu
from jax.experimental.pallas import tpu_sc as plsc

NUM_ENTITIES = 1000000
EMBED_DIM = 64
BATCH = 16384

NC, NS, L = 2, 16, 16
NW = NC * NS
B_PER_W = BATCH // NW
CHUNK = 128
N_CHUNKS = B_PER_W // CHUNK


def _body(h_hbm, r_hbm, t_hbm, er_hbm, ei_hbm, rr_hbm, ri_hbm, out_hbm,
          h_i, r_i, t_i, hr_b, hi_b, tr_b, ti_b, rr_b, ri_b,
          psum_b, score_b, sem):
    wid = lax.axis_index("s") * NC + lax.axis_index("c")
    base = wid * B_PER_W
    lanes = lax.iota(jnp.int32, L)

    for c in range(N_CHUNKS):
        cbase = base + c * CHUNK
        pltpu.sync_copy(h_hbm.at[pl.ds(cbase, CHUNK)], h_i)
        pltpu.sync_copy(r_hbm.at[pl.ds(cbase, CHUNK)], r_i)
        pltpu.sync_copy(t_hbm.at[pl.ds(cbase, CHUNK)], t_i)

        cps = [
            pltpu.async_copy(er_hbm.at[h_i], hr_b, sem),
            pltpu.async_copy(ei_hbm.at[h_i], hi_b, sem),
            pltpu.async_copy(er_hbm.at[t_i], tr_b, sem),
            pltpu.async_copy(ei_hbm.at[t_i], ti_b, sem),
            pltpu.async_copy(rr_hbm.at[r_i], rr_b, sem),
            pltpu.async_copy(ri_hbm.at[r_i], ri_b, sem),
        ]
        for cp in cps:
            cp.wait()

        def row_step(i, carry):
            acc = None
            for j in range(EMBED_DIM // L):
                s = pl.ds(j * L, L)
                vhr = hr_b[i, s]
                vhi = hi_b[i, s]
                vtr = tr_b[i, s]
                vti = ti_b[i, s]
                vrr = rr_b[i, s]
                vri = ri_b[i, s]
                p1 = vhr * vtr + vhi * vti
                p2 = vhi * vtr - vhr * vti
                term = (1.0 + vrr) * p1 - vri * p2
                acc = term if acc is None else acc + term
            psum_b[pl.ds(i * L, L)] = acc
            return carry

        lax.fori_loop(0, CHUNK, row_step, 0, unroll=4)

        for g in range(CHUNK // L):
            rows = g * L + lanes
            acc = None
            for d in range(L):
                idx = rows * L + d
                v = plsc.load_gather(psum_b, [idx])
                acc = v if acc is None else acc + v
            score_b[pl.ds(c * CHUNK + g * L, L)] = acc

    pltpu.sync_copy(score_b, out_hbm.at[pl.ds(base, B_PER_W)])


@jax.jit
def _complex_score(h, r, t, ent_real, ent_imag, rel_real, rel_imag):
    mesh = plsc.VectorSubcoreMesh(core_axis_name="c", subcore_axis_name="s")
    kern = pl.kernel(
        _body,
        out_type=jax.ShapeDtypeStruct((BATCH,), jnp.float32),
        mesh=mesh,
        compiler_params=pltpu.CompilerParams(
            needs_layout_passes=False, use_tc_tiling_on_sc=False),
        scratch_types=[
            pltpu.VMEM((CHUNK,), jnp.int32),
            pltpu.VMEM((CHUNK,), jnp.int32),
            pltpu.VMEM((CHUNK,), jnp.int32),
            pltpu.VMEM((CHUNK, EMBED_DIM), jnp.float32),
            pltpu.VMEM((CHUNK, EMBED_DIM), jnp.float32),
            pltpu.VMEM((CHUNK, EMBED_DIM), jnp.float32),
            pltpu.VMEM((CHUNK, EMBED_DIM), jnp.float32),
            pltpu.VMEM((CHUNK, EMBED_DIM), jnp.float32),
            pltpu.VMEM((CHUNK, EMBED_DIM), jnp.float32),
            pltpu.VMEM((CHUNK * L,), jnp.float32),
            pltpu.VMEM((B_PER_W,), jnp.float32),
            pltpu.SemaphoreType.DMA,
        ],
    )
    return kern(h, r, t, ent_real, ent_imag, rel_real, rel_imag)


def kernel(h, r, t, ent_real, ent_imag, rel_real, rel_imag):
    h = h.astype(jnp.int32)
    r = r.astype(jnp.int32)
    t = t.astype(jnp.int32)
    return _complex_score(h, r, t, ent_real, ent_imag, rel_real, rel_imag)


# speedup vs baseline: 1.3159x; 1.3159x over previous
import jax
import jax.numpy as jnp
from jax import lax
from jax.experimental import pallas as pl
from jax.experimental.pallas import tpu as pltpu
from jax.experimental.pallas import tpu_sc as plsc

NUM_ENTITIES = 1000000
EMBED_DIM = 64
BATCH = 16384

NC, NS, L = 2, 16, 16
NW = NC * NS
B_PER_W = BATCH // NW
CHUNK = 8
N_CHUNKS = B_PER_W // CHUNK
TILE = 8


def _body(h_hbm, r_hbm, t_hbm, er_hbm, ei_hbm, rr_hbm, ri_hbm, out_hbm,
          h_s0, r_s0, t_s0, h_s1, r_s1, t_s1,
          g00, g01, g02, g03, g04, g05,
          g10, g11, g12, g13, g14, g15,
          psum_b, score_b, ibounce, sem0, sem1):
    idx = [(h_s0, r_s0, t_s0), (h_s1, r_s1, t_s1)]
    bufs = [(g00, g01, g02, g03, g04, g05), (g10, g11, g12, g13, g14, g15)]
    sems = [sem0, sem1]

    wid = lax.axis_index("s") * NC + lax.axis_index("c")
    base = wid * B_PER_W
    lanes = lax.iota(jnp.int32, L)

    def to_smem(hbm, cbase, sm_ref):
        pltpu.sync_copy(hbm.at[pl.ds(cbase, CHUNK)], ibounce.at[pl.ds(0, CHUNK)])
        v = ibounce[pl.ds(0, L)]
        for j in range(CHUNK):
            sm_ref[j] = jnp.sum(jnp.where(lanes == j, v, 0))

    def issue(cbase, k):
        h_s, r_s, t_s = idx[k]
        to_smem(h_hbm, cbase, h_s)
        to_smem(r_hbm, cbase, r_s)
        to_smem(t_hbm, cbase, t_s)
        hr_b, hi_b, tr_b, ti_b, rr_b, ri_b = bufs[k]
        sem = sems[k]
        for i in range(CHUNK):
            hb = (h_s[i] // TILE) * TILE
            rb = (r_s[i] // TILE) * TILE
            tb = (t_s[i] // TILE) * TILE
            d = pl.ds(i * TILE, TILE)
            pltpu.async_copy(er_hbm.at[pl.ds(hb, TILE)], hr_b.at[d], sem)
            pltpu.async_copy(ei_hbm.at[pl.ds(hb, TILE)], hi_b.at[d], sem)
            pltpu.async_copy(er_hbm.at[pl.ds(tb, TILE)], tr_b.at[d], sem)
            pltpu.async_copy(ei_hbm.at[pl.ds(tb, TILE)], ti_b.at[d], sem)
            pltpu.async_copy(rr_hbm.at[pl.ds(rb, TILE)], rr_b.at[d], sem)
            pltpu.async_copy(ri_hbm.at[pl.ds(rb, TILE)], ri_b.at[d], sem)

    def drain(k):
        sem = sems[k]
        src = rr_hbm.at[pl.ds(0, CHUNK * TILE)]
        for buf in bufs[k]:
            pltpu.make_async_copy(src, buf, sem).wait()

    def compute(cidx, k):
        h_s, r_s, t_s = idx[k]
        hr_b, hi_b, tr_b, ti_b, rr_b, ri_b = bufs[k]
        for i in range(CHUNK):
            hm = lax.rem(h_s[i], TILE) + i * TILE
            rm = lax.rem(r_s[i], TILE) + i * TILE
            tm = lax.rem(t_s[i], TILE) + i * TILE
            acc = None
            for j in range(EMBED_DIM // L):
                s = pl.ds(j * L, L)
                vhr = hr_b[hm, s]
                vhi = hi_b[hm, s]
                vtr = tr_b[tm, s]
                vti = ti_b[tm, s]
                vrr = rr_b[rm, s]
                vri = ri_b[rm, s]
                p1 = vhr * vtr + vhi * vti
                p2 = vhi * vtr - vhr * vti
                term = (1.0 + vrr) * p1 - vri * p2
                acc = term if acc is None else acc + term
            psum_b[pl.ds((cidx * CHUNK + i) * L, L)] = acc

    issue(base, 0)
    issue(base + CHUNK, 1)

    def step(it, carry):
        c0 = 2 * it
        drain(0)
        compute(c0, 0)

        @pl.when(c0 + 2 < N_CHUNKS)
        def _():
            issue(base + (c0 + 2) * CHUNK, 0)

        drain(1)
        compute(c0 + 1, 1)

        @pl.when(c0 + 3 < N_CHUNKS)
        def _():
            issue(base + (c0 + 3) * CHUNK, 1)

        return carry

    lax.fori_loop(0, N_CHUNKS // 2, step, 0)

    for g in range(B_PER_W // L):
        rows = g * L + lanes
        acc = None
        for d in range(L):
            fidx = rows * L + d
            v = plsc.load_gather(psum_b, [fidx])
            acc = v if acc is None else acc + v
        score_b[pl.ds(g * L, L)] = acc

    pltpu.sync_copy(score_b, out_hbm.at[pl.ds(base, B_PER_W)])


@jax.jit
def _complex_score(h, r, t, ent_real, ent_imag, rel_real, rel_imag):
    mesh = plsc.VectorSubcoreMesh(core_axis_name="c", subcore_axis_name="s")
    ibuf = pltpu.SMEM((CHUNK,), jnp.int32)
    gbuf = pltpu.VMEM((CHUNK * TILE, EMBED_DIM), jnp.float32)
    kern = pl.kernel(
        _body,
        out_type=jax.ShapeDtypeStruct((BATCH,), jnp.float32),
        mesh=mesh,
        compiler_params=pltpu.CompilerParams(needs_layout_passes=False),
        scratch_types=[
            ibuf, ibuf, ibuf, ibuf, ibuf, ibuf,
            gbuf, gbuf, gbuf, gbuf, gbuf, gbuf,
            gbuf, gbuf, gbuf, gbuf, gbuf, gbuf,
            pltpu.VMEM((B_PER_W * L,), jnp.float32),
            pltpu.VMEM((B_PER_W,), jnp.float32),
            pltpu.VMEM((L,), jnp.int32),
            pltpu.SemaphoreType.DMA,
            pltpu.SemaphoreType.DMA,
        ],
    )
    return kern(h, r, t, ent_real, ent_imag, rel_real, rel_imag)


def kernel(h, r, t, ent_real, ent_imag, rel_real, rel_imag):
    h = h.astype(jnp.int32)
    r = r.astype(jnp.int32)
    t = t.astype(jnp.int32)
    return _complex_score(h, r, t, ent_real, ent_imag, rel_real, rel_imag)
